# baseline (speedup 1.0000x reference)
import jax
import jax.numpy as jnp
from jax import lax
from jax.experimental import pallas as pl
from jax.experimental.pallas import tpu as pltpu
from jax.experimental.pallas import tpu_sc as plsc

_D = 64
_B = 4096
_H = 200
_N = _B * _H
_NW = 32
_PER_W = _N // _NW
_CH = 128
_NCHUNK = _PER_W // _CH
_NBUF = 4
_NGROUP = _NCHUNK // _NBUF


def _body(table, idx, out, idxb, rows, *sems):
    gsem = sems[:_NBUF]
    psem = sems[_NBUF:]
    w = lax.axis_index("s") * 2 + lax.axis_index("c")
    base = w * _PER_W

    pltpu.sync_copy(idx.at[w], idxb)

    for b in range(_NBUF):
        pltpu.async_copy(table.at[idxb.at[b]], rows.at[b], gsem[b])

    @pl.loop(0, _NGROUP - 1)
    def _group(g):
        for b in range(_NBUF):
            j = g * _NBUF + b
            o = out.at[pl.ds(base + j * _CH, _CH)]
            pltpu.make_async_copy(
                table.at[idxb.at[j]], rows.at[b], gsem[b]).wait()
            pltpu.async_copy(rows.at[b], o, psem[b])
            pltpu.make_async_copy(rows.at[b], o, psem[b]).wait()
            pltpu.async_copy(table.at[idxb.at[j + _NBUF]], rows.at[b], gsem[b])

    for b in range(_NBUF):
        j = (_NGROUP - 1) * _NBUF + b
        o = out.at[pl.ds(base + j * _CH, _CH)]
        pltpu.make_async_copy(table.at[idxb.at[j]], rows.at[b], gsem[b]).wait()
        pltpu.async_copy(rows.at[b], o, psem[b])
    for b in range(_NBUF):
        j = (_NGROUP - 1) * _NBUF + b
        o = out.at[pl.ds(base + j * _CH, _CH)]
        pltpu.make_async_copy(rows.at[b], o, psem[b]).wait()


_TW = 2048


def _tp_body(tt_ref, out_ref):
    blk = tt_ref[...]
    eye = jnp.eye(64, dtype=jnp.float32)
    t = lax.dot_general(blk, eye, (((0,), (0,)), ((), ())),
                        preferred_element_type=jnp.float32)
    out_ref[:, :64] = t


_tc_pad = pl.pallas_call(
    _tp_body,
    out_shape=jax.ShapeDtypeStruct((1000000, 128), jnp.float32),
    grid=(pl.cdiv(1000000, _TW),),
    in_specs=[pl.BlockSpec((64, _TW), lambda i: (0, i))],
    out_specs=pl.BlockSpec((_TW, 128), lambda i: (i, 0)),
)


_mesh = plsc.VectorSubcoreMesh(core_axis_name="c", subcore_axis_name="s")

_gather = pl.kernel(
    _body,
    out_type=jax.ShapeDtypeStruct((_N, 128), jnp.float32),
    mesh=_mesh,
    scratch_types=[
        pltpu.VMEM((_NCHUNK, _CH), jnp.int32),
        pltpu.VMEM((_NBUF, _CH, 128), jnp.float32),
    ] + [pltpu.SemaphoreType.DMA] * (2 * _NBUF),
)


@jax.jit
def kernel(x, embedding):
    idx = x.astype(jnp.int32).reshape(_NW, _NCHUNK, _CH)
    table = _tc_pad(jnp.transpose(embedding))
    out = _gather(table, idx)
    return out.reshape(_B, _H, 128)[:, :, :_D]


# speedup vs baseline: 1.0199x; 1.0199x over previous
import jax
import jax.numpy as jnp
from jax import lax
from jax.experimental import pallas as pl
from jax.experimental.pallas import tpu as pltpu
from jax.experimental.pallas import tpu_sc as plsc

_D = 64
_B = 4096
_H = 200
_N = _B * _H
_NW = 32
_PER_W = _N // _NW
_CH = 128
_NCHUNK = _PER_W // _CH
_NBUF = 4
_NGROUP = _NCHUNK // _NBUF


def _body(table, idx, out, idxb, rows, *sems):
    gsem = sems[:_NBUF]
    psem = sems[_NBUF:]
    w = lax.axis_index("s") * 2 + lax.axis_index("c")
    base = w * _PER_W

    pltpu.sync_copy(idx.at[w], idxb)

    for b in range(_NBUF):
        pltpu.async_copy(table.at[idxb.at[b]], rows.at[b], gsem[b])

    @pl.loop(0, _NGROUP - 1)
    def _group(g):
        for b in range(_NBUF):
            j = g * _NBUF + b
            o = out.at[pl.ds(base + j * _CH, _CH)]
            pltpu.make_async_copy(
                table.at[idxb.at[j]], rows.at[b], gsem[b]).wait()
            pltpu.async_copy(rows.at[b], o, psem[b])
            pltpu.make_async_copy(rows.at[b], o, psem[b]).wait()
            pltpu.async_copy(table.at[idxb.at[j + _NBUF]], rows.at[b], gsem[b])

    for b in range(_NBUF):
        j = (_NGROUP - 1) * _NBUF + b
        o = out.at[pl.ds(base + j * _CH, _CH)]
        pltpu.make_async_copy(table.at[idxb.at[j]], rows.at[b], gsem[b]).wait()
        pltpu.async_copy(rows.at[b], o, psem[b])
    for b in range(_NBUF):
        j = (_NGROUP - 1) * _NBUF + b
        o = out.at[pl.ds(base + j * _CH, _CH)]
        pltpu.make_async_copy(rows.at[b], o, psem[b]).wait()


_TW = 2048


def _tp_body(tt_ref, out_ref):
    blk = tt_ref[...]
    t = jnp.transpose(blk, (1, 0))
    out_ref[:, :64] = t


_tc_pad = pl.pallas_call(
    _tp_body,
    out_shape=jax.ShapeDtypeStruct((1000000, 128), jnp.float32),
    grid=(pl.cdiv(1000000, _TW),),
    in_specs=[pl.BlockSpec((64, _TW), lambda i: (0, i))],
    out_specs=pl.BlockSpec((_TW, 128), lambda i: (i, 0)),
)


_mesh = plsc.VectorSubcoreMesh(core_axis_name="c", subcore_axis_name="s")

_gather = pl.kernel(
    _body,
    out_type=jax.ShapeDtypeStruct((_N, 128), jnp.float32),
    mesh=_mesh,
    scratch_types=[
        pltpu.VMEM((_NCHUNK, _CH), jnp.int32),
        pltpu.VMEM((_NBUF, _CH, 128), jnp.float32),
    ] + [pltpu.SemaphoreType.DMA] * (2 * _NBUF),
)


@jax.jit
def kernel(x, embedding):
    idx = x.astype(jnp.int32).reshape(_NW, _NCHUNK, _CH)
    table = _tc_pad(jnp.transpose(embedding))
    out = _gather(table, idx)
    return out.reshape(_B, _H, 128)[:, :, :_D]
